# Initial kernel scaffold; baseline (speedup 1.0000x reference)
import functools

import jax
import jax.numpy as jnp
from jax import lax
from jax.experimental import pallas as pl
from jax.experimental.pallas import tpu as pltpu
from jax.experimental.pallas import tpu_sc as plsc

NC = 2
NS = 16
NW = NC * NS
CH = 128
L = 16


def _rsqrt16(d):
    di = plsc.bitcast(d, jnp.int32)
    u = plsc.bitcast(jnp.int32(0x5F3759DF) - lax.shift_right_logical(di, 1),
                     jnp.float32)
    for _ in range(3):
        u = u * (1.5 - 0.5 * d * u * u)
    return u


def _make_deg_kernel(n, nch):
    mesh = plsc.VectorSubcoreMesh(core_axis_name="c", subcore_axis_name="s")
    stripe = 640
    last = n - (NS - 1) * stripe

    @functools.partial(
        pl.kernel,
        out_type=jax.ShapeDtypeStruct((NC, n), jnp.float32),
        mesh=mesh,
        scratch_types=[
            pltpu.VMEM_SHARED((n,), jnp.float32),
            pltpu.VMEM((nch, CH), jnp.int32),
            pltpu.VMEM((nch, CH), jnp.float32),
            pltpu.VMEM((stripe,), jnp.float32),
        ],
    )
    def deg_kernel(col_hbm, ew_hbm, out_hbm, acc, col_v, ew_v, zb):
        cid = lax.axis_index("c")
        sid = lax.axis_index("s")
        wid = cid * NS + sid

        @pl.loop(0, stripe // L)
        def _(i):
            zb[pl.ds(i * L, L)] = jnp.zeros((L,), jnp.float32)

        @pl.when(sid < NS - 1)
        def _():
            pltpu.sync_copy(zb, acc.at[pl.ds(sid * stripe, stripe)])

        @pl.when(sid == NS - 1)
        def _():
            pltpu.sync_copy(zb.at[pl.ds(0, last)],
                            acc.at[pl.ds(sid * stripe, last)])

        pltpu.sync_copy(col_hbm.at[wid], col_v)
        pltpu.sync_copy(ew_hbm.at[wid], ew_v)
        plsc.subcore_barrier()

        @pl.loop(0, nch)
        def _(j):
            pltpu.sync_copy(ew_v.at[j], acc.at[col_v.at[j]], add=True)

        plsc.subcore_barrier()

        @pl.when(sid < NS - 1)
        def _():
            pltpu.sync_copy(acc.at[pl.ds(sid * stripe, stripe)],
                            out_hbm.at[cid, pl.ds(sid * stripe, stripe)])

        @pl.when(sid == NS - 1)
        def _():
            pltpu.sync_copy(acc.at[pl.ds(sid * stripe, last)],
                            out_hbm.at[cid, pl.ds(sid * stripe, last)])

    return deg_kernel


def _make_agg_kernel(n, d, nch):
    mesh = plsc.VectorSubcoreMesh(core_axis_name="c", subcore_axis_name="s")
    rows_per_tile = n // NS
    drain = 125
    ndrain = rows_per_tile // drain

    @functools.partial(
        pl.kernel,
        out_type=jax.ShapeDtypeStruct((NC, n, d), jnp.float32),
        mesh=mesh,
        scratch_types=[
            pltpu.VMEM_SHARED((n, d), jnp.float32),
            pltpu.VMEM((nch, CH), jnp.int32),
            pltpu.VMEM((nch, CH), jnp.int32),
            pltpu.VMEM((nch, CH), jnp.float32),
            pltpu.VMEM((n,), jnp.float32),
            pltpu.VMEM((CH, d), jnp.float32),
            pltpu.VMEM((CH,), jnp.float32),
            pltpu.VMEM((drain, d), jnp.float32),
            pltpu.VMEM((drain, d), jnp.float32),
        ],
    )
    def agg_kernel(x_hbm, row_hbm, col_hbm, ew_hbm, deg_hbm, out_hbm,
                   acc, row_v, col_v, ew_v, dis_v, gb, sb, db, xb):
        cid = lax.axis_index("c")
        sid = lax.axis_index("s")
        wid = cid * NS + sid
        base_row = sid * rows_per_tile

        pltpu.sync_copy(row_hbm.at[wid], row_v)
        pltpu.sync_copy(col_hbm.at[wid], col_v)
        pltpu.sync_copy(ew_hbm.at[wid], ew_v)
        pltpu.sync_copy(deg_hbm, dis_v)

        @pl.loop(0, n // L)
        def _(i):
            sl = pl.ds(i * L, L)
            dis_v[sl] = _rsqrt16(dis_v[sl])

        @pl.loop(0, CH)
        def _(e):
            for dd in range(d // L):
                gb[e, pl.ds(dd * L, L)] = jnp.zeros((L,), jnp.float32)

        for k in range(ndrain):
            pltpu.sync_copy(
                gb.at[pl.ds(0, drain)],
                acc.at[pl.ds(base_row + k * drain, drain)])
        plsc.subcore_barrier()

        @pl.loop(0, nch)
        def _(j):
            pltpu.sync_copy(x_hbm.at[row_v.at[j]], gb)
            for g in range(CH // L):
                sl = pl.ds(g * L, L)
                r16 = row_v[j, sl]
                c16 = col_v[j, sl]
                w16 = ew_v[j, sl]
                dr = plsc.load_gather(dis_v, [r16])
                dc = plsc.load_gather(dis_v, [c16])
                sb[sl] = w16 * dr * dc

            @pl.loop(0, CH)
            def _(e):
                s = sb[e]
                for dd in range(d // L):
                    dsl = pl.ds(dd * L, L)
                    gb[e, dsl] = gb[e, dsl] * s

            pltpu.sync_copy(gb, acc.at[col_v.at[j]], add=True)

        plsc.subcore_barrier()

        for k in range(ndrain):
            r0 = base_row + k * drain
            pltpu.sync_copy(acc.at[pl.ds(r0, drain)], db)

            @pl.when(cid == 0)
            def _():
                pltpu.sync_copy(x_hbm.at[pl.ds(r0, drain)], xb)

                @pl.loop(0, drain)
                def _(r):
                    s = dis_v[r0 + r]
                    s2 = s * s
                    for dd in range(d // L):
                        dsl = pl.ds(dd * L, L)
                        db[r, dsl] = db[r, dsl] + xb[r, dsl] * s2

            pltpu.sync_copy(db, out_hbm.at[cid, pl.ds(r0, drain)])

    return agg_kernel


def _final_tc(agg, w, b2, n, d, rb):
    def body(a_ref, w_ref, b_ref, o_ref):
        a = a_ref[0] + a_ref[1]
        o_ref[...] = jnp.dot(a, w_ref[...],
                             preferred_element_type=jnp.float32) + b_ref[...]

    return pl.pallas_call(
        body,
        grid=(n // rb,),
        in_specs=[
            pl.BlockSpec((NC, rb, d), lambda i: (0, i, 0)),
            pl.BlockSpec((d, d), lambda i: (0, 0)),
            pl.BlockSpec((1, d), lambda i: (0, 0)),
        ],
        out_specs=pl.BlockSpec((rb, d), lambda i: (i, 0)),
        out_shape=jax.ShapeDtypeStruct((n, d), jnp.float32),
    )(agg, w, b2)


def kernel(x, edge_index, edge_weight, W, b):
    n, d = x.shape
    e = edge_index.shape[1]
    ept = -(-e // NW)
    nch = -(-ept // CH)
    if nch % 2:
        nch += 1
    e_pad = NW * nch * CH

    pad = e_pad - e
    row = jnp.concatenate(
        [edge_index[0], jnp.zeros((pad,), jnp.int32)]).reshape(NW, nch, CH)
    col = jnp.concatenate(
        [edge_index[1], jnp.zeros((pad,), jnp.int32)]).reshape(NW, nch, CH)
    ew = jnp.concatenate(
        [edge_weight, jnp.zeros((pad,), jnp.float32)]).reshape(NW, nch, CH)

    deg_parts = _make_deg_kernel(n, nch)(col, ew)
    deg = deg_parts[0] + deg_parts[1] + 1.0
    agg = _make_agg_kernel(n, d, nch)(x, row, col, ew, deg)
    return _final_tc(agg, W, b.reshape(1, d), n, d, 1000)


# speedup vs baseline: 14.6639x; 14.6639x over previous
import dataclasses
import functools

import jax
import jax.numpy as jnp
from jax import lax
from jax.experimental import pallas as pl
from jax.experimental.pallas import tpu as pltpu
from jax.experimental.pallas import tpu_sc as plsc

NC = 2
NS = 16
NW = NC * NS
CH = 128
L = 16


def _sc_params():
    cp = pltpu.CompilerParams()
    if "needs_layout_passes" in pltpu.CompilerParams.__dataclass_fields__:
        cp = dataclasses.replace(cp, needs_layout_passes=False)
    return cp


def _rsqrt16(d):
    di = plsc.bitcast(d, jnp.int32)
    u = plsc.bitcast(jnp.int32(0x5F3759DF) - lax.shift_right_logical(di, 1),
                     jnp.float32)
    for _ in range(3):
        u = u * (1.5 - 0.5 * d * u * u)
    return u


def _make_deg_kernel(n, nch):
    mesh = plsc.VectorSubcoreMesh(core_axis_name="c", subcore_axis_name="s")
    stripe = 640
    last = n - (NS - 1) * stripe

    @functools.partial(
        pl.kernel,
        out_type=jax.ShapeDtypeStruct((NC * n,), jnp.float32),
        mesh=mesh,
        scratch_types=[
            pltpu.VMEM_SHARED((n,), jnp.float32),
            pltpu.VMEM((nch, CH), jnp.int32),
            pltpu.VMEM((nch, CH), jnp.float32),
            pltpu.VMEM((stripe,), jnp.float32),
        ],
    )
    def deg_kernel(col_hbm, ew_hbm, out_hbm, acc, col_v, ew_v, zb):
        cid = lax.axis_index("c")
        sid = lax.axis_index("s")
        wid = cid * NS + sid

        @pl.loop(0, stripe // L)
        def _(i):
            zb[pl.ds(i * L, L)] = jnp.zeros((L,), jnp.float32)

        @pl.when(sid < NS - 1)
        def _():
            pltpu.sync_copy(zb, acc.at[pl.ds(sid * stripe, stripe)])

        @pl.when(sid == NS - 1)
        def _():
            pltpu.sync_copy(zb.at[pl.ds(0, last)],
                            acc.at[pl.ds(sid * stripe, last)])

        pltpu.sync_copy(col_hbm.at[wid], col_v)
        pltpu.sync_copy(ew_hbm.at[wid], ew_v)
        plsc.subcore_barrier()

        @pl.loop(0, nch)
        def _(j):
            pltpu.sync_copy(ew_v.at[j], acc.at[col_v.at[j]], add=True)

        plsc.subcore_barrier()

        @pl.when(sid < NS - 1)
        def _():
            pltpu.sync_copy(acc.at[pl.ds(sid * stripe, stripe)], zb)
            pltpu.sync_copy(zb,
                            out_hbm.at[pl.ds(cid * n + sid * stripe, stripe)])

        @pl.when(sid == NS - 1)
        def _():
            pltpu.sync_copy(acc.at[pl.ds(sid * stripe, last)],
                            zb.at[pl.ds(0, last)])
            pltpu.sync_copy(zb.at[pl.ds(0, last)],
                            out_hbm.at[pl.ds(cid * n + sid * stripe, last)])

    return deg_kernel


def _make_agg_kernel(n, d, nch):
    mesh = plsc.VectorSubcoreMesh(core_axis_name="c", subcore_axis_name="s")
    stripe = 640
    last = n - (NS - 1) * stripe
    drain = 128
    last_full = last // drain
    last_rem = last - last_full * drain

    blk = 16
    nblk = nch // blk
    rem = nch - nblk * blk

    @functools.partial(
        pl.kernel,
        out_type=jax.ShapeDtypeStruct((NC, n, d), jnp.float32),
        mesh=mesh,
        compiler_params=_sc_params(),
        scratch_types=[
            pltpu.VMEM_SHARED((n, d), jnp.float32),
            pltpu.VMEM((blk, CH), jnp.int32),
            pltpu.VMEM((blk, CH), jnp.int32),
            pltpu.VMEM((blk, CH), jnp.float32),
            pltpu.VMEM((n,), jnp.float32),
            pltpu.VMEM((CH, d), jnp.float32),
        ],
    )
    def agg_kernel(x_hbm, row_hbm, col_hbm, ew_hbm, deg_hbm, out_hbm,
                   acc, row_v, col_v, ew_v, dis_v, gb):
        db = gb
        cid = lax.axis_index("c")
        sid = lax.axis_index("s")
        wid = cid * NS + sid
        base_row = sid * stripe

        pltpu.sync_copy(deg_hbm, dis_v)

        @pl.loop(0, n // L)
        def _(i):
            sl = pl.ds(i * L, L)
            dis_v[sl] = _rsqrt16(dis_v[sl])

        @pl.loop(0, CH)
        def _(e):
            for dd in range(d // L):
                gb[e, pl.ds(dd * L, L)] = jnp.zeros((L,), jnp.float32)

        @pl.when(sid < NS - 1)
        def _():
            for k in range(stripe // drain):
                pltpu.sync_copy(gb, acc.at[pl.ds(base_row + k * drain, drain)])

        @pl.when(sid == NS - 1)
        def _():
            for k in range(last_full):
                pltpu.sync_copy(gb, acc.at[pl.ds(base_row + k * drain, drain)])
            if last_rem:
                pltpu.sync_copy(
                    gb.at[pl.ds(0, last_rem)],
                    acc.at[pl.ds(base_row + last_full * drain, last_rem)])
        plsc.subcore_barrier()

        def process_chunk(j):
            pltpu.sync_copy(x_hbm.at[row_v.at[j]], gb)

            @pl.loop(0, CH // L)
            def _(g):
                sl = pl.ds(g * L, L)
                r16 = row_v[j, sl]
                c16 = col_v[j, sl]
                w16 = ew_v[j, sl]
                dr = plsc.load_gather(dis_v, [r16])
                dc = plsc.load_gather(dis_v, [c16])
                s16 = w16 * dr * dc
                for ee in range(L):
                    s = s16[ee]
                    row_i = g * L + ee
                    for dd in range(d // L):
                        dsl = pl.ds(dd * L, L)
                        gb[row_i, dsl] = gb[row_i, dsl] * s

            pltpu.sync_copy(gb, acc.at[col_v.at[j]], add=True)

        def stage_block(ofs, count):
            sl_h = pl.ds(ofs, count)
            sl_v = pl.ds(0, count)
            pltpu.sync_copy(row_hbm.at[wid, sl_h], row_v.at[sl_v])
            pltpu.sync_copy(col_hbm.at[wid, sl_h], col_v.at[sl_v])
            pltpu.sync_copy(ew_hbm.at[wid, sl_h], ew_v.at[sl_v])

        @pl.loop(0, nblk)
        def _(bk):
            stage_block(pl.multiple_of(bk * blk, blk), blk)

            @pl.loop(0, blk)
            def _(j):
                process_chunk(j)

        if rem:
            stage_block(nblk * blk, rem)

            @pl.loop(0, rem)
            def _(j):
                process_chunk(j)

        plsc.subcore_barrier()

        @pl.when(sid < NS - 1)
        def _():
            for k in range(stripe // drain):
                r0 = base_row + k * drain
                pltpu.sync_copy(acc.at[pl.ds(r0, drain)], db)
                pltpu.sync_copy(db, out_hbm.at[cid, pl.ds(r0, drain)])

        @pl.when(sid == NS - 1)
        def _():
            for k in range(last_full):
                r0 = base_row + k * drain
                pltpu.sync_copy(acc.at[pl.ds(r0, drain)], db)
                pltpu.sync_copy(db, out_hbm.at[cid, pl.ds(r0, drain)])
            if last_rem:
                r0 = base_row + last_full * drain
                pltpu.sync_copy(acc.at[pl.ds(r0, last_rem)],
                                db.at[pl.ds(0, last_rem)])
                pltpu.sync_copy(db.at[pl.ds(0, last_rem)],
                                out_hbm.at[cid, pl.ds(r0, last_rem)])

    return agg_kernel


def _final_tc(agg, w, b2, n, d, rb):
    def body(a_ref, w_ref, b_ref, o_ref):
        a = a_ref[0] + a_ref[1]
        o_ref[...] = jnp.dot(a, w_ref[...],
                             preferred_element_type=jnp.float32) + b_ref[...]

    return pl.pallas_call(
        body,
        grid=(n // rb,),
        in_specs=[
            pl.BlockSpec((NC, rb, d), lambda i: (0, i, 0)),
            pl.BlockSpec((d, d), lambda i: (0, 0)),
            pl.BlockSpec((1, d), lambda i: (0, 0)),
        ],
        out_specs=pl.BlockSpec((rb, d), lambda i: (i, 0)),
        out_shape=jax.ShapeDtypeStruct((n, d), jnp.float32),
    )(agg, w, b2)


def kernel(x, edge_index, edge_weight, W, b):
    n, d = x.shape
    e = edge_index.shape[1] + n
    ept = -(-e // NW)
    nch = -(-ept // CH)
    if nch % 2:
        nch += 1
    e_pad = NW * nch * CH

    pad = e_pad - e
    loop_idx = jnp.arange(n, dtype=jnp.int32)
    zpad_i = jnp.zeros((pad,), jnp.int32)
    row = jnp.concatenate(
        [edge_index[0], loop_idx, zpad_i]).reshape(NW, nch, CH)
    col = jnp.concatenate(
        [edge_index[1], loop_idx, zpad_i]).reshape(NW, nch, CH)
    ew = jnp.concatenate(
        [edge_weight, jnp.ones((n,), jnp.float32),
         jnp.zeros((pad,), jnp.float32)]).reshape(NW, nch, CH)

    deg_parts = _make_deg_kernel(n, nch)(col, ew).reshape(NC, n)
    deg = deg_parts[0] + deg_parts[1]
    agg = _make_agg_kernel(n, d, nch)(x, row, col, ew, deg)
    return _final_tc(agg, W, b.reshape(1, d), n, d, 1000)


# speedup vs baseline: 17.8334x; 1.2161x over previous
import dataclasses
import functools

import jax
import jax.numpy as jnp
from jax import lax
from jax.experimental import pallas as pl
from jax.experimental.pallas import tpu as pltpu
from jax.experimental.pallas import tpu_sc as plsc

NC = 2
NS = 16
NW = NC * NS
CH = 128
L = 16


def _sc_params():
    cp = pltpu.CompilerParams()
    if "needs_layout_passes" in pltpu.CompilerParams.__dataclass_fields__:
        cp = dataclasses.replace(cp, needs_layout_passes=False)
    return cp


def _rsqrt16(d):
    di = plsc.bitcast(d, jnp.int32)
    u = plsc.bitcast(jnp.int32(0x5F3759DF) - lax.shift_right_logical(di, 1),
                     jnp.float32)
    for _ in range(3):
        u = u * (1.5 - 0.5 * d * u * u)
    return u


def _make_deg_kernel(n, nch):
    mesh = plsc.VectorSubcoreMesh(core_axis_name="c", subcore_axis_name="s")
    stripe = 640
    last = n - (NS - 1) * stripe

    @functools.partial(
        pl.kernel,
        out_type=jax.ShapeDtypeStruct((NC * n,), jnp.float32),
        mesh=mesh,
        scratch_types=[
            pltpu.VMEM_SHARED((n,), jnp.float32),
            pltpu.VMEM((nch, CH), jnp.int32),
            pltpu.VMEM((nch, CH), jnp.float32),
            pltpu.VMEM((stripe,), jnp.float32),
        ],
    )
    def deg_kernel(col_hbm, ew_hbm, out_hbm, acc, col_v, ew_v, zb):
        cid = lax.axis_index("c")
        sid = lax.axis_index("s")
        wid = cid * NS + sid

        @pl.loop(0, stripe // L)
        def _(i):
            zb[pl.ds(i * L, L)] = jnp.zeros((L,), jnp.float32)

        @pl.when(sid < NS - 1)
        def _():
            pltpu.sync_copy(zb, acc.at[pl.ds(sid * stripe, stripe)])

        @pl.when(sid == NS - 1)
        def _():
            pltpu.sync_copy(zb.at[pl.ds(0, last)],
                            acc.at[pl.ds(sid * stripe, last)])

        pltpu.sync_copy(col_hbm.at[wid], col_v)
        pltpu.sync_copy(ew_hbm.at[wid], ew_v)
        plsc.subcore_barrier()

        @pl.loop(0, nch)
        def _(j):
            pltpu.sync_copy(ew_v.at[j], acc.at[col_v.at[j]], add=True)

        plsc.subcore_barrier()

        @pl.when(sid < NS - 1)
        def _():
            pltpu.sync_copy(acc.at[pl.ds(sid * stripe, stripe)], zb)
            pltpu.sync_copy(zb,
                            out_hbm.at[pl.ds(cid * n + sid * stripe, stripe)])

        @pl.when(sid == NS - 1)
        def _():
            pltpu.sync_copy(acc.at[pl.ds(sid * stripe, last)],
                            zb.at[pl.ds(0, last)])
            pltpu.sync_copy(zb.at[pl.ds(0, last)],
                            out_hbm.at[pl.ds(cid * n + sid * stripe, last)])

    return deg_kernel


def _make_agg_kernel(n, d, nch):
    mesh = plsc.VectorSubcoreMesh(core_axis_name="c", subcore_axis_name="s")
    stripe = 640
    last = n - (NS - 1) * stripe
    drain = 128
    last_full = last // drain
    last_rem = last - last_full * drain

    blk = 16
    nblk = nch // blk
    rem = nch - nblk * blk

    @functools.partial(
        pl.kernel,
        out_type=jax.ShapeDtypeStruct((NC, n, d), jnp.float32),
        mesh=mesh,
        compiler_params=_sc_params(),
        scratch_types=[
            pltpu.VMEM_SHARED((n, d), jnp.float32),
            pltpu.VMEM((blk, CH), jnp.int32),
            pltpu.VMEM((blk, CH), jnp.int32),
            pltpu.VMEM((blk, CH), jnp.float32),
            pltpu.VMEM((n,), jnp.float32),
            pltpu.VMEM((CH, d), jnp.float32),
            pltpu.VMEM((CH, d), jnp.float32),
            pltpu.SemaphoreType.DMA,
            pltpu.SemaphoreType.DMA,
            pltpu.SemaphoreType.DMA,
            pltpu.SemaphoreType.DMA,
        ],
    )
    def agg_kernel(x_hbm, row_hbm, col_hbm, ew_hbm, deg_hbm, out_hbm,
                   acc, row_v, col_v, ew_v, dis_v, gb, gb1, sg0, sg1, ss0, ss1):
        db = gb
        cid = lax.axis_index("c")
        sid = lax.axis_index("s")
        wid = cid * NS + sid
        base_row = sid * stripe

        pltpu.sync_copy(deg_hbm, dis_v)

        @pl.loop(0, n // L)
        def _(i):
            sl = pl.ds(i * L, L)
            dis_v[sl] = _rsqrt16(dis_v[sl])

        @pl.loop(0, CH)
        def _(e):
            for dd in range(d // L):
                gb[e, pl.ds(dd * L, L)] = jnp.zeros((L,), jnp.float32)

        @pl.when(sid < NS - 1)
        def _():
            for k in range(stripe // drain):
                pltpu.sync_copy(gb, acc.at[pl.ds(base_row + k * drain, drain)])

        @pl.when(sid == NS - 1)
        def _():
            for k in range(last_full):
                pltpu.sync_copy(gb, acc.at[pl.ds(base_row + k * drain, drain)])
            if last_rem:
                pltpu.sync_copy(
                    gb.at[pl.ds(0, last_rem)],
                    acc.at[pl.ds(base_row + last_full * drain, last_rem)])
        plsc.subcore_barrier()

        def scale(buf, j):
            @pl.loop(0, CH // L)
            def _(g):
                sl = pl.ds(g * L, L)
                r16 = row_v[j, sl]
                c16 = col_v[j, sl]
                w16 = ew_v[j, sl]
                dr = plsc.load_gather(dis_v, [r16])
                dc = plsc.load_gather(dis_v, [c16])
                s16 = w16 * dr * dc
                for ee in range(L):
                    s = s16[ee]
                    row_i = g * L + ee
                    for dd in range(d // L):
                        dsl = pl.ds(dd * L, L)
                        buf[row_i, dsl] = buf[row_i, dsl] * s

        def g_start(j, buf, sem):
            pltpu.async_copy(x_hbm.at[row_v.at[j]], buf, sem)

        def g_wait(buf, sem):
            pltpu.make_async_copy(x_hbm.at[row_v.at[0]], buf, sem).wait()

        def s_start(j, buf, sem):
            pltpu.async_copy(buf, acc.at[col_v.at[j]], sem, add=True)

        def s_wait(buf, sem):
            pltpu.make_async_copy(buf, acc.at[col_v.at[0]], sem).wait()

        def stage_block(ofs, count):
            sl_h = pl.ds(ofs, count)
            sl_v = pl.ds(0, count)
            pltpu.sync_copy(row_hbm.at[wid, sl_h], row_v.at[sl_v])
            pltpu.sync_copy(col_hbm.at[wid, sl_h], col_v.at[sl_v])
            pltpu.sync_copy(ew_hbm.at[wid, sl_h], ew_v.at[sl_v])

        def run_block(count):
            half = count // 2
            g_start(0, gb, sg0)

            @pl.loop(0, half)
            def _(t):
                j0 = t * 2
                j1 = j0 + 1

                @pl.when(t > 0)
                def _():
                    s_wait(gb1, ss1)

                g_start(j1, gb1, sg1)
                g_wait(gb, sg0)
                scale(gb, j0)
                s_start(j0, gb, ss0)

                @pl.when(t < half - 1)
                def _():
                    s_wait(gb, ss0)
                    g_start(j0 + 2, gb, sg0)

                g_wait(gb1, sg1)
                scale(gb1, j1)
                s_start(j1, gb1, ss1)

            s_wait(gb, ss0)
            s_wait(gb1, ss1)

        @pl.loop(0, nblk)
        def _(bk):
            stage_block(pl.multiple_of(bk * blk, blk), blk)
            run_block(blk)

        if rem:
            stage_block(nblk * blk, rem)
            run_block(rem)

        plsc.subcore_barrier()

        @pl.when(sid < NS - 1)
        def _():
            for k in range(stripe // drain):
                r0 = base_row + k * drain
                pltpu.sync_copy(acc.at[pl.ds(r0, drain)], db)
                pltpu.sync_copy(db, out_hbm.at[cid, pl.ds(r0, drain)])

        @pl.when(sid == NS - 1)
        def _():
            for k in range(last_full):
                r0 = base_row + k * drain
                pltpu.sync_copy(acc.at[pl.ds(r0, drain)], db)
                pltpu.sync_copy(db, out_hbm.at[cid, pl.ds(r0, drain)])
            if last_rem:
                r0 = base_row + last_full * drain
                pltpu.sync_copy(acc.at[pl.ds(r0, last_rem)],
                                db.at[pl.ds(0, last_rem)])
                pltpu.sync_copy(db.at[pl.ds(0, last_rem)],
                                out_hbm.at[cid, pl.ds(r0, last_rem)])

    return agg_kernel


def _final_tc(agg, w, b2, n, d, rb):
    def body(a_ref, w_ref, b_ref, o_ref):
        a = a_ref[0] + a_ref[1]
        o_ref[...] = jnp.dot(a, w_ref[...],
                             preferred_element_type=jnp.float32) + b_ref[...]

    return pl.pallas_call(
        body,
        grid=(n // rb,),
        in_specs=[
            pl.BlockSpec((NC, rb, d), lambda i: (0, i, 0)),
            pl.BlockSpec((d, d), lambda i: (0, 0)),
            pl.BlockSpec((1, d), lambda i: (0, 0)),
        ],
        out_specs=pl.BlockSpec((rb, d), lambda i: (i, 0)),
        out_shape=jax.ShapeDtypeStruct((n, d), jnp.float32),
    )(agg, w, b2)


def kernel(x, edge_index, edge_weight, W, b):
    n, d = x.shape
    e = edge_index.shape[1] + n
    ept = -(-e // NW)
    nch = -(-ept // CH)
    if nch % 2:
        nch += 1
    e_pad = NW * nch * CH

    pad = e_pad - e
    loop_idx = jnp.arange(n, dtype=jnp.int32)
    zpad_i = jnp.zeros((pad,), jnp.int32)
    row = jnp.concatenate(
        [edge_index[0], loop_idx, zpad_i]).reshape(NW, nch, CH)
    col = jnp.concatenate(
        [edge_index[1], loop_idx, zpad_i]).reshape(NW, nch, CH)
    ew = jnp.concatenate(
        [edge_weight, jnp.ones((n,), jnp.float32),
         jnp.zeros((pad,), jnp.float32)]).reshape(NW, nch, CH)

    deg_parts = _make_deg_kernel(n, nch)(col, ew).reshape(NC, n)
    deg = deg_parts[0] + deg_parts[1]
    agg = _make_agg_kernel(n, d, nch)(x, row, col, ew, deg)
    return _final_tc(agg, W, b.reshape(1, d), n, d, 1000)


# speedup vs baseline: 32.4269x; 1.8183x over previous
import dataclasses
import functools

import jax
import jax.numpy as jnp
from jax import lax
from jax.experimental import pallas as pl
from jax.experimental.pallas import tpu as pltpu
from jax.experimental.pallas import tpu_sc as plsc

NC = 2
NS = 16
NW = NC * NS
CH = 128
L = 16


def _sc_params():
    cp = pltpu.CompilerParams()
    if "needs_layout_passes" in pltpu.CompilerParams.__dataclass_fields__:
        cp = dataclasses.replace(cp, needs_layout_passes=False)
    return cp


def _rsqrt16(d):
    di = plsc.bitcast(d, jnp.int32)
    u = plsc.bitcast(jnp.int32(0x5F3759DF) - lax.shift_right_logical(di, 1),
                     jnp.float32)
    for _ in range(3):
        u = u * (1.5 - 0.5 * d * u * u)
    return u


def _make_deg_kernel(n, nch):
    mesh = plsc.VectorSubcoreMesh(core_axis_name="c", subcore_axis_name="s")
    stripe = 640
    last = n - (NS - 1) * stripe

    @functools.partial(
        pl.kernel,
        out_type=jax.ShapeDtypeStruct((NC * n,), jnp.float32),
        mesh=mesh,
        scratch_types=[
            pltpu.VMEM_SHARED((n,), jnp.float32),
            pltpu.VMEM((nch, CH), jnp.int32),
            pltpu.VMEM((nch, CH), jnp.float32),
            pltpu.VMEM((stripe,), jnp.float32),
        ],
    )
    def deg_kernel(col_hbm, ew_hbm, out_hbm, acc, col_v, ew_v, zb):
        cid = lax.axis_index("c")
        sid = lax.axis_index("s")
        wid = cid * NS + sid

        @pl.loop(0, stripe // L)
        def _(i):
            zb[pl.ds(i * L, L)] = jnp.zeros((L,), jnp.float32)

        @pl.when(sid < NS - 1)
        def _():
            pltpu.sync_copy(zb, acc.at[pl.ds(sid * stripe, stripe)])

        @pl.when(sid == NS - 1)
        def _():
            pltpu.sync_copy(zb.at[pl.ds(0, last)],
                            acc.at[pl.ds(sid * stripe, last)])

        pltpu.sync_copy(col_hbm.at[wid], col_v)
        pltpu.sync_copy(ew_hbm.at[wid], ew_v)
        plsc.subcore_barrier()

        @pl.loop(0, nch)
        def _(j):
            pltpu.sync_copy(ew_v.at[j], acc.at[col_v.at[j]], add=True)

        plsc.subcore_barrier()

        @pl.when(sid < NS - 1)
        def _():
            pltpu.sync_copy(acc.at[pl.ds(sid * stripe, stripe)], zb)
            pltpu.sync_copy(zb,
                            out_hbm.at[pl.ds(cid * n + sid * stripe, stripe)])

        @pl.when(sid == NS - 1)
        def _():
            pltpu.sync_copy(acc.at[pl.ds(sid * stripe, last)],
                            zb.at[pl.ds(0, last)])
            pltpu.sync_copy(zb.at[pl.ds(0, last)],
                            out_hbm.at[pl.ds(cid * n + sid * stripe, last)])

    return deg_kernel


def _make_agg_kernel(n, d, nch):
    mesh = plsc.VectorSubcoreMesh(core_axis_name="c", subcore_axis_name="s")
    stripe = 640
    last = n - (NS - 1) * stripe
    drain = 128
    last_full = last // drain
    last_rem = last - last_full * drain

    blk = 16
    nblk = nch // blk
    rem = nch - nblk * blk

    @functools.partial(
        pl.kernel,
        out_type=jax.ShapeDtypeStruct((NC, n, d), jnp.float32),
        mesh=mesh,
        compiler_params=_sc_params(),
        scratch_types=[
            pltpu.VMEM_SHARED((n, d), jnp.float32),
            pltpu.VMEM((blk, CH), jnp.int32),
            pltpu.VMEM((blk, CH), jnp.int32),
            pltpu.VMEM((blk, CH), jnp.float32),
            pltpu.VMEM((n,), jnp.float32),
            pltpu.VMEM((CH, d), jnp.float32),
            pltpu.VMEM((CH, d), jnp.float32),
            pltpu.SemaphoreType.DMA,
            pltpu.SemaphoreType.DMA,
            pltpu.SemaphoreType.DMA,
            pltpu.SemaphoreType.DMA,
        ],
    )
    def agg_kernel(x_hbm, row_hbm, col_hbm, ew_hbm, deg_hbm, out_hbm,
                   acc, row_v, col_v, ew_v, dis_v, gb, gb1, sg0, sg1, ss0, ss1):
        db = gb
        cid = lax.axis_index("c")
        sid = lax.axis_index("s")
        wid = cid * NS + sid
        base_row = sid * stripe

        pltpu.sync_copy(deg_hbm, dis_v)

        @pl.loop(0, n // L)
        def _(i):
            sl = pl.ds(i * L, L)
            dis_v[sl] = _rsqrt16(dis_v[sl])

        @pl.loop(0, CH)
        def _(e):
            for dd in range(d // L):
                gb[e, pl.ds(dd * L, L)] = jnp.zeros((L,), jnp.float32)

        @pl.when(sid < NS - 1)
        def _():
            for k in range(stripe // drain):
                pltpu.sync_copy(gb, acc.at[pl.ds(base_row + k * drain, drain)])

        @pl.when(sid == NS - 1)
        def _():
            for k in range(last_full):
                pltpu.sync_copy(gb, acc.at[pl.ds(base_row + k * drain, drain)])
            if last_rem:
                pltpu.sync_copy(
                    gb.at[pl.ds(0, last_rem)],
                    acc.at[pl.ds(base_row + last_full * drain, last_rem)])
        plsc.subcore_barrier()

        def scale(buf, j):
            @pl.loop(0, CH // L)
            def _(g):
                sl = pl.ds(g * L, L)
                r16 = row_v[j, sl]
                c16 = col_v[j, sl]
                w16 = ew_v[j, sl]
                dr = plsc.load_gather(dis_v, [r16])
                dc = plsc.load_gather(dis_v, [c16])
                s16 = w16 * dr * dc
                for ee in range(L):
                    s = s16[ee]
                    row_i = g * L + ee
                    for dd in range(d // L):
                        dsl = pl.ds(dd * L, L)
                        buf[row_i, dsl] = buf[row_i, dsl] * s

        def g_start(j, buf, sem):
            pltpu.async_copy(x_hbm.at[row_v.at[j]], buf, sem)

        def g_wait(buf, sem):
            pltpu.make_async_copy(x_hbm.at[row_v.at[0]], buf, sem).wait()

        def s_start(j, buf, sem):
            pltpu.async_copy(buf, acc.at[col_v.at[j]], sem, add=True)

        def s_wait(buf, sem):
            pltpu.make_async_copy(buf, acc.at[col_v.at[0]], sem).wait()

        def stage_block(ofs, count):
            sl_h = pl.ds(ofs, count)
            sl_v = pl.ds(0, count)
            pltpu.sync_copy(row_hbm.at[wid, sl_h], row_v.at[sl_v])
            pltpu.sync_copy(col_hbm.at[wid, sl_h], col_v.at[sl_v])
            pltpu.sync_copy(ew_hbm.at[wid, sl_h], ew_v.at[sl_v])

        def run_block(count):
            half = count // 2
            g_start(0, gb, sg0)

            @pl.loop(0, half)
            def _(t):
                j0 = t * 2
                j1 = j0 + 1

                @pl.when(t > 0)
                def _():
                    s_wait(gb1, ss1)

                g_start(j1, gb1, sg1)
                g_wait(gb, sg0)
                scale(gb, j0)
                s_start(j0, gb, ss0)

                @pl.when(t < half - 1)
                def _():
                    s_wait(gb, ss0)
                    g_start(j0 + 2, gb, sg0)

                g_wait(gb1, sg1)
                scale(gb1, j1)
                s_start(j1, gb1, ss1)

            s_wait(gb, ss0)
            s_wait(gb1, ss1)

        @pl.loop(0, nblk)
        def _(bk):
            stage_block(pl.multiple_of(bk * blk, blk), blk)
            run_block(blk)

        if rem:
            stage_block(nblk * blk, rem)
            run_block(rem)

        plsc.subcore_barrier()

        @pl.when(sid < NS - 1)
        def _():
            for k in range(stripe // drain):
                r0 = base_row + k * drain
                pltpu.sync_copy(acc.at[pl.ds(r0, drain)], db)
                pltpu.sync_copy(db, out_hbm.at[cid, pl.ds(r0, drain)])

        @pl.when(sid == NS - 1)
        def _():
            for k in range(last_full):
                r0 = base_row + k * drain
                pltpu.sync_copy(acc.at[pl.ds(r0, drain)], db)
                pltpu.sync_copy(db, out_hbm.at[cid, pl.ds(r0, drain)])
            if last_rem:
                r0 = base_row + last_full * drain
                pltpu.sync_copy(acc.at[pl.ds(r0, last_rem)],
                                db.at[pl.ds(0, last_rem)])
                pltpu.sync_copy(db.at[pl.ds(0, last_rem)],
                                out_hbm.at[cid, pl.ds(r0, last_rem)])

    return agg_kernel


def _final_tc(agg, w, b2, n, d, rb):
    def body(a_ref, w_ref, b_ref, o_ref):
        a = a_ref[0] + a_ref[1]
        o_ref[...] = jnp.dot(a, w_ref[...],
                             preferred_element_type=jnp.float32) + b_ref[...]

    return pl.pallas_call(
        body,
        grid=(n // rb,),
        in_specs=[
            pl.BlockSpec((NC, rb, d), lambda i: (0, i, 0)),
            pl.BlockSpec((d, d), lambda i: (0, 0)),
            pl.BlockSpec((1, d), lambda i: (0, 0)),
        ],
        out_specs=pl.BlockSpec((rb, d), lambda i: (i, 0)),
        out_shape=jax.ShapeDtypeStruct((n, d), jnp.float32),
    )(agg, w, b2)


def kernel(x, edge_index, edge_weight, W, b):
    n, d = x.shape
    e = edge_index.shape[1] + n
    ept = -(-e // NW)
    nch = -(-ept // CH)
    if nch % 2:
        nch += 1
    e_pad = NW * nch * CH

    pad = e_pad - e
    loop_idx = jnp.arange(n, dtype=jnp.int32)
    zpad_i = jnp.arange(pad, dtype=jnp.int32) % n
    row = jnp.concatenate(
        [edge_index[0], loop_idx, zpad_i]).reshape(NW, nch, CH)
    col = jnp.concatenate(
        [edge_index[1], loop_idx, zpad_i]).reshape(NW, nch, CH)
    ew = jnp.concatenate(
        [edge_weight, jnp.ones((n,), jnp.float32),
         jnp.zeros((pad,), jnp.float32)]).reshape(NW, nch, CH)

    deg_parts = _make_deg_kernel(n, nch)(col, ew).reshape(NC, n)
    deg = deg_parts[0] + deg_parts[1]
    agg = _make_agg_kernel(n, d, nch)(x, row, col, ew, deg)
    return _final_tc(agg, W, b.reshape(1, d), n, d, 1000)
